# baseline (speedup 1.0000x reference)
import functools

import jax
import jax.numpy as jnp
from jax import lax
from jax.experimental import pallas as pl
from jax.experimental.pallas import tpu as pltpu
from jax.experimental.pallas import tpu_sc as plsc

DIM = 768
HID = DIM * 4
E = 16
T = 2048
TILE = 256
ALIGN = 8
PAD = T + E * ALIGN + TILE - ALIGN
PAD = ((PAD + TILE - 1) // TILE) * TILE
NH = 4
HC = HID // NH
MROW = 16
MCOL = T // MROW


def _routing_body(mt_ref, pos_ref, start_ref, cnt_ref):
    mt = mt_ref[...]
    acc = jnp.zeros((MROW, MCOL), jnp.int32)
    ps = jnp.int32(0)
    for e in range(E):
        m = (mt == e).astype(jnp.int32)
        p = m
        s = 1
        while s < MCOL:
            p = p + jnp.concatenate(
                [jnp.zeros((MROW, s), jnp.int32), p[:, :MCOL - s]], axis=1)
            s *= 2
        rt = p[:, MCOL - 1:MCOL]
        q = rt
        s = 1
        while s < MROW:
            q = q + jnp.concatenate(
                [jnp.zeros((s, 1), jnp.int32), q[:MROW - s, :]], axis=0)
            s *= 2
        rank = p + (q - rt)
        acc = acc + m * (ps + rank - 1)
        cnt = jnp.sum(m)
        start_ref[e] = ps
        cnt_ref[e] = cnt
        ps = ps + ((cnt + ALIGN - 1) // ALIGN) * ALIGN
    pos_ref[...] = acc


def _routing(mt2d):
    return pl.pallas_call(
        _routing_body,
        out_shape=[
            jax.ShapeDtypeStruct((MROW, MCOL), jnp.int32),
            jax.ShapeDtypeStruct((E,), jnp.int32),
            jax.ShapeDtypeStruct((E,), jnp.int32),
        ],
        out_specs=[
            pl.BlockSpec(memory_space=pltpu.MemorySpace.VMEM),
            pl.BlockSpec(memory_space=pltpu.SMEM),
            pl.BlockSpec(memory_space=pltpu.SMEM),
        ],
        in_specs=[pl.BlockSpec(memory_space=pltpu.MemorySpace.VMEM)],
    )(mt2d)


def _make_sc_row_perm(n_src, n_dst, dim, scatter):
    info = plsc.get_sparse_core_info()
    nc, ns = info.num_cores, info.num_subcores
    nw = nc * ns
    per_w = n_src // nw
    mesh = plsc.VectorSubcoreMesh(core_axis_name="c", subcore_axis_name="s")

    @functools.partial(
        pl.kernel,
        mesh=mesh,
        out_type=jax.ShapeDtypeStruct((n_dst, dim), jnp.float32),
        scratch_types=[
            pltpu.VMEM((per_w,), jnp.int32),
            pltpu.VMEM((per_w, dim), jnp.float32),
            pltpu.SemaphoreType.DMA,
        ],
    )
    def perm_k(rows_hbm, idx_hbm, out_hbm, idx_v, rows_v, sem):
        wid = lax.axis_index("s") * nc + lax.axis_index("c")
        base = wid * per_w
        pltpu.sync_copy(idx_hbm.at[pl.ds(base, per_w)], idx_v)
        if scatter:
            pltpu.sync_copy(rows_hbm.at[pl.ds(base, per_w)], rows_v)
            pltpu.async_copy(rows_v, out_hbm.at[idx_v], sem).wait()
        else:
            pltpu.async_copy(rows_hbm.at[idx_v], rows_v, sem).wait()
            pltpu.sync_copy(rows_v, out_hbm.at[pl.ds(base, per_w)])

    return perm_k


def _ffn_body(start_ref, cnt_ref,
              x_ref, wg_ref, bg_ref, wi_ref, bi_ref, wo_ref, bo_ref,
              out_ref):
    e = pl.program_id(0)
    h = pl.program_id(1)
    start = start_ref[e]
    cnt = cnt_ref[e]
    ntiles = (cnt + TILE - 1) // TILE
    wg = wg_ref[0]
    wi = wi_ref[0]
    wo = wo_ref[0]
    bg = bg_ref[0, 0]
    bi = bi_ref[0, 0]
    bo = bo_ref[0, 0]

    def tile_body(k, _):
        base = pl.multiple_of(start, ALIGN) + k * TILE
        xb = x_ref[pl.ds(base, TILE), :]
        g = jnp.dot(xb, wg, preferred_element_type=jnp.float32) + bg
        i_ = jnp.dot(xb, wi, preferred_element_type=jnp.float32) + bi
        hdn = (g * jax.nn.sigmoid(g)) * i_
        o = jnp.dot(hdn, wo, preferred_element_type=jnp.float32)
        rows = k * TILE + lax.broadcasted_iota(jnp.int32, (TILE, 1), 0)
        mask = rows < cnt
        bo_term = jnp.where(h == 0, bo, 0.0)
        prev = jnp.where(h == 0, 0.0, out_ref[pl.ds(base, TILE), :])
        out_ref[pl.ds(base, TILE), :] = prev + jnp.where(mask, o + bo_term, 0.0)
        return 0

    lax.fori_loop(0, ntiles, tile_body, 0)


def _grouped_ffn(start_w, cnt_w, x_pad, Wg, bg, Wi, bi, Wo, bo):
    grid_spec = pltpu.PrefetchScalarGridSpec(
        num_scalar_prefetch=2,
        grid=(E, NH),
        in_specs=[
            pl.BlockSpec((PAD, DIM), lambda e, h, st, cn: (0, 0)),
            pl.BlockSpec((1, DIM, HC), lambda e, h, st, cn: (e, 0, h)),
            pl.BlockSpec((1, 1, HC), lambda e, h, st, cn: (e, 0, h)),
            pl.BlockSpec((1, DIM, HC), lambda e, h, st, cn: (e, 0, h)),
            pl.BlockSpec((1, 1, HC), lambda e, h, st, cn: (e, 0, h)),
            pl.BlockSpec((1, HC, DIM), lambda e, h, st, cn: (e, h, 0)),
            pl.BlockSpec((1, 1, DIM), lambda e, h, st, cn: (e, 0, 0)),
        ],
        out_specs=pl.BlockSpec((PAD, DIM), lambda e, h, st, cn: (0, 0)),
    )
    return pl.pallas_call(
        _ffn_body,
        grid_spec=grid_spec,
        out_shape=jax.ShapeDtypeStruct((PAD, DIM), jnp.float32),
    )(start_w, cnt_w, x_pad,
      Wg, bg.reshape(E, 1, HID), Wi, bi.reshape(E, 1, HID),
      Wo, bo.reshape(E, 1, DIM))


def kernel(x, mapped_tokens, Wg, bg, Wi, bi, Wo, bo):
    Bv, Tv, C = x.shape
    xf = x.reshape(Bv * Tv, C)
    mt2d = mapped_tokens.astype(jnp.int32).reshape(MROW, MCOL)

    pos2d, start_w, cnt_w = _routing(mt2d)
    pos = pos2d.reshape(T)

    x_pad = _make_sc_row_perm(T, PAD, DIM, scatter=True)(xf, pos)
    out_pad = _grouped_ffn(start_w, cnt_w, x_pad, Wg, bg, Wi, bi, Wo, bo)
    out = _make_sc_row_perm(T, T, DIM, scatter=False)(out_pad, pos)
    return out.reshape(Bv, Tv, C)


# speedup vs baseline: 1.0815x; 1.0815x over previous
import functools

import jax
import jax.numpy as jnp
from jax import lax
from jax.experimental import pallas as pl
from jax.experimental.pallas import tpu as pltpu
from jax.experimental.pallas import tpu_sc as plsc

DIM = 768
HID = DIM * 4
E = 16
T = 2048
TILE = 256
ALIGN = 8
PAD = T + E * ALIGN + TILE - ALIGN
PAD = ((PAD + TILE - 1) // TILE) * TILE
NH = 2
HC = HID // NH
HC2 = HC // 2
MROW = 16
MCOL = T // MROW


def _routing_body(mt_ref, pos_ref, start_ref, cnt_ref):
    mt = mt_ref[...]
    acc = jnp.zeros((MROW, MCOL), jnp.int32)
    ps = jnp.int32(0)
    for e in range(E):
        m = (mt == e).astype(jnp.int32)
        p = m
        s = 1
        while s < MCOL:
            p = p + jnp.concatenate(
                [jnp.zeros((MROW, s), jnp.int32), p[:, :MCOL - s]], axis=1)
            s *= 2
        rt = p[:, MCOL - 1:MCOL]
        q = rt
        s = 1
        while s < MROW:
            q = q + jnp.concatenate(
                [jnp.zeros((s, 1), jnp.int32), q[:MROW - s, :]], axis=0)
            s *= 2
        rank = p + (q - rt)
        acc = acc + m * (ps + rank - 1)
        cnt = jnp.sum(m)
        start_ref[e] = ps
        cnt_ref[e] = cnt
        ps = ps + ((cnt + ALIGN - 1) // ALIGN) * ALIGN
    pos_ref[...] = acc


def _routing(mt2d):
    return pl.pallas_call(
        _routing_body,
        out_shape=[
            jax.ShapeDtypeStruct((MROW, MCOL), jnp.int32),
            jax.ShapeDtypeStruct((E,), jnp.int32),
            jax.ShapeDtypeStruct((E,), jnp.int32),
        ],
        out_specs=[
            pl.BlockSpec(memory_space=pltpu.MemorySpace.VMEM),
            pl.BlockSpec(memory_space=pltpu.SMEM),
            pl.BlockSpec(memory_space=pltpu.SMEM),
        ],
        in_specs=[pl.BlockSpec(memory_space=pltpu.MemorySpace.VMEM)],
    )(mt2d)


def _make_sc_row_perm(n_src, n_dst, dim, scatter):
    info = plsc.get_sparse_core_info()
    nc, ns = info.num_cores, info.num_subcores
    nw = nc * ns
    per_w = n_src // nw
    mesh = plsc.VectorSubcoreMesh(core_axis_name="c", subcore_axis_name="s")

    @functools.partial(
        pl.kernel,
        mesh=mesh,
        out_type=jax.ShapeDtypeStruct((n_dst, dim), jnp.float32),
        scratch_types=[
            pltpu.VMEM((per_w,), jnp.int32),
            pltpu.VMEM((per_w, dim), jnp.float32),
            pltpu.SemaphoreType.DMA,
        ],
    )
    def perm_k(rows_hbm, idx_hbm, out_hbm, idx_v, rows_v, sem):
        wid = lax.axis_index("s") * nc + lax.axis_index("c")
        base = wid * per_w
        pltpu.sync_copy(idx_hbm.at[pl.ds(base, per_w)], idx_v)
        if scatter:
            pltpu.sync_copy(rows_hbm.at[pl.ds(base, per_w)], rows_v)
            pltpu.async_copy(rows_v, out_hbm.at[idx_v], sem).wait()
        else:
            pltpu.async_copy(rows_hbm.at[idx_v], rows_v, sem).wait()
            pltpu.sync_copy(rows_v, out_hbm.at[pl.ds(base, per_w)])

    return perm_k


def _ffn_body(start_ref, cnt_ref,
              x_ref, wg0_ref, wg1_ref, bg_ref, wi0_ref, wi1_ref, bi_ref,
              wo0_ref, wo1_ref, bo_ref,
              out_ref):
    e = pl.program_id(0)
    h = pl.program_id(1)
    start = start_ref[e]
    cnt = cnt_ref[e]
    ntiles = (cnt + TILE - 1) // TILE
    halves = ((wg0_ref[0], wi0_ref[0], wo0_ref[0]),
              (wg1_ref[0], wi1_ref[0], wo1_ref[0]))
    bg = bg_ref[0, 0]
    bi = bi_ref[0, 0]
    bo = bo_ref[0, 0]

    def tile_body(k, _):
        base = pl.multiple_of(start, ALIGN) + k * TILE
        xb = x_ref[pl.ds(base, TILE), :]
        o = None
        for j, (wg, wi, wo) in enumerate(halves):
            g = jnp.dot(xb, wg, preferred_element_type=jnp.float32)
            g = g + bg[j * HC2:(j + 1) * HC2]
            i_ = jnp.dot(xb, wi, preferred_element_type=jnp.float32)
            i_ = i_ + bi[j * HC2:(j + 1) * HC2]
            hdn = (g * jax.nn.sigmoid(g)) * i_
            oj = jnp.dot(hdn, wo, preferred_element_type=jnp.float32)
            o = oj if o is None else o + oj
        rows = k * TILE + lax.broadcasted_iota(jnp.int32, (TILE, 1), 0)
        mask = rows < cnt
        bo_term = jnp.where(h == 0, bo, 0.0)
        prev = jnp.where(h == 0, 0.0, out_ref[pl.ds(base, TILE), :])
        out_ref[pl.ds(base, TILE), :] = prev + jnp.where(mask, o + bo_term, 0.0)
        return 0

    lax.fori_loop(0, ntiles, tile_body, 0)


def _grouped_ffn(start_w, cnt_w, x_pad, Wg, bg, Wi, bi, Wo, bo):
    whalf = lambda j: pl.BlockSpec(
        (1, DIM, HC2), lambda e, h, st, cn, j=j: (e, 0, 2 * h + j))
    wohalf = lambda j: pl.BlockSpec(
        (1, HC2, DIM), lambda e, h, st, cn, j=j: (e, 2 * h + j, 0))
    grid_spec = pltpu.PrefetchScalarGridSpec(
        num_scalar_prefetch=2,
        grid=(E, NH),
        in_specs=[
            pl.BlockSpec((PAD, DIM), lambda e, h, st, cn: (0, 0)),
            whalf(0), whalf(1),
            pl.BlockSpec((1, 1, HC), lambda e, h, st, cn: (e, 0, h)),
            whalf(0), whalf(1),
            pl.BlockSpec((1, 1, HC), lambda e, h, st, cn: (e, 0, h)),
            wohalf(0), wohalf(1),
            pl.BlockSpec((1, 1, DIM), lambda e, h, st, cn: (e, 0, 0)),
        ],
        out_specs=pl.BlockSpec((PAD, DIM), lambda e, h, st, cn: (0, 0)),
    )
    return pl.pallas_call(
        _ffn_body,
        grid_spec=grid_spec,
        out_shape=jax.ShapeDtypeStruct((PAD, DIM), jnp.float32),
    )(start_w, cnt_w, x_pad,
      Wg, Wg, bg.reshape(E, 1, HID), Wi, Wi, bi.reshape(E, 1, HID),
      Wo, Wo, bo.reshape(E, 1, DIM))


def kernel(x, mapped_tokens, Wg, bg, Wi, bi, Wo, bo):
    Bv, Tv, C = x.shape
    xf = x.reshape(Bv * Tv, C)
    mt2d = mapped_tokens.astype(jnp.int32).reshape(MROW, MCOL)

    pos2d, start_w, cnt_w = _routing(mt2d)
    pos = pos2d.reshape(T)

    x_pad = _make_sc_row_perm(T, PAD, DIM, scatter=True)(xf, pos)
    out_pad = _grouped_ffn(start_w, cnt_w, x_pad, Wg, bg, Wi, bi, Wo, bo)
    out = _make_sc_row_perm(T, T, DIM, scatter=False)(out_pad, pos)
    return out.reshape(Bv, Tv, C)


# speedup vs baseline: 1.0866x; 1.0048x over previous
import functools

import jax
import jax.numpy as jnp
from jax import lax
from jax.experimental import pallas as pl
from jax.experimental.pallas import tpu as pltpu
from jax.experimental.pallas import tpu_sc as plsc

DIM = 768
HID = DIM * 4
E = 16
T = 2048
TILE = 256
ALIGN = 8
PAD = T + E * ALIGN + TILE - ALIGN
PAD = ((PAD + TILE - 1) // TILE) * TILE
NH = 2
HC = HID // NH
HC2 = HC // 2
MROW = 16
MCOL = T // MROW


def _routing_body(mt_ref, pos_ref, start_ref, cnt_ref):
    mt = mt_ref[...]
    acc = jnp.zeros((MROW, MCOL), jnp.int32)
    ps = jnp.int32(0)
    for e in range(E):
        m = (mt == e).astype(jnp.int32)
        p = m
        s = 1
        while s < MCOL:
            p = p + jnp.concatenate(
                [jnp.zeros((MROW, s), jnp.int32), p[:, :MCOL - s]], axis=1)
            s *= 2
        rt = p[:, MCOL - 1:MCOL]
        q = rt
        s = 1
        while s < MROW:
            q = q + jnp.concatenate(
                [jnp.zeros((s, 1), jnp.int32), q[:MROW - s, :]], axis=0)
            s *= 2
        rank = p + (q - rt)
        acc = acc + m * (ps + rank - 1)
        cnt = jnp.sum(m)
        start_ref[e] = ps
        cnt_ref[e] = cnt
        ps = ps + ((cnt + ALIGN - 1) // ALIGN) * ALIGN
    pos_ref[...] = acc


def _routing(mt2d):
    return pl.pallas_call(
        _routing_body,
        out_shape=[
            jax.ShapeDtypeStruct((MROW, MCOL), jnp.int32),
            jax.ShapeDtypeStruct((E,), jnp.int32),
            jax.ShapeDtypeStruct((E,), jnp.int32),
        ],
        out_specs=[
            pl.BlockSpec(memory_space=pltpu.MemorySpace.VMEM),
            pl.BlockSpec(memory_space=pltpu.SMEM),
            pl.BlockSpec(memory_space=pltpu.SMEM),
        ],
        in_specs=[pl.BlockSpec(memory_space=pltpu.MemorySpace.VMEM)],
    )(mt2d)


def _make_sc_row_perm(n_src, n_dst, dim, scatter):
    info = plsc.get_sparse_core_info()
    nc, ns = info.num_cores, info.num_subcores
    nw = nc * ns
    per_w = n_src // nw
    mesh = plsc.VectorSubcoreMesh(core_axis_name="c", subcore_axis_name="s")

    @functools.partial(
        pl.kernel,
        mesh=mesh,
        out_type=jax.ShapeDtypeStruct((n_dst, dim), jnp.float32),
        scratch_types=[
            pltpu.VMEM((per_w,), jnp.int32),
            pltpu.VMEM((per_w, dim), jnp.float32),
            pltpu.SemaphoreType.DMA,
        ],
    )
    def perm_k(rows_hbm, idx_hbm, out_hbm, idx_v, rows_v, sem):
        wid = lax.axis_index("s") * nc + lax.axis_index("c")
        base = wid * per_w
        pltpu.sync_copy(idx_hbm.at[pl.ds(base, per_w)], idx_v)
        if scatter:
            pltpu.sync_copy(rows_hbm.at[pl.ds(base, per_w)], rows_v)
            pltpu.async_copy(rows_v, out_hbm.at[idx_v], sem).wait()
        else:
            pltpu.async_copy(rows_hbm.at[idx_v], rows_v, sem).wait()
            pltpu.sync_copy(rows_v, out_hbm.at[pl.ds(base, per_w)])

    return perm_k


def _ffn_body(start_ref, cnt_ref,
              x_ref, wg_ref, bg_ref, wi_ref, bi_ref, wo_ref, bo_ref,
              out_ref):
    e = pl.program_id(0)
    h = pl.program_id(1)
    start = start_ref[e]
    cnt = cnt_ref[e]
    ntiles = (cnt + TILE - 1) // TILE
    wg = wg_ref[0]
    wi = wi_ref[0]
    wo = wo_ref[0]
    bg = bg_ref[0, 0]
    bi = bi_ref[0, 0]
    bo = bo_ref[0, 0]

    def tile_body(k, _):
        base = pl.multiple_of(start, ALIGN) + k * TILE
        xb = x_ref[pl.ds(base, TILE), :]
        g = jnp.dot(xb, wg, preferred_element_type=jnp.float32) + bg
        i_ = jnp.dot(xb, wi, preferred_element_type=jnp.float32) + bi
        hdn = (g * jax.nn.sigmoid(g)) * i_
        o = jnp.dot(hdn, wo, preferred_element_type=jnp.float32)
        rows = k * TILE + lax.broadcasted_iota(jnp.int32, (TILE, 1), 0)
        mask = rows < cnt

        @pl.when(h == 0)
        def _first():
            out_ref[pl.ds(base, TILE), :] = jnp.where(mask, o + bo, 0.0)

        @pl.when(h != 0)
        def _acc():
            out_ref[pl.ds(base, TILE), :] += jnp.where(mask, o, 0.0)

        return 0

    lax.fori_loop(0, ntiles, tile_body, 0)


def _grouped_ffn(start_w, cnt_w, x_pad, Wg, bg, Wi, bi, Wo, bo):
    grid_spec = pltpu.PrefetchScalarGridSpec(
        num_scalar_prefetch=2,
        grid=(E, NH),
        in_specs=[
            pl.BlockSpec((PAD, DIM), lambda e, h, st, cn: (0, 0)),
            pl.BlockSpec((1, DIM, HC), lambda e, h, st, cn: (e, 0, h)),
            pl.BlockSpec((1, 1, HC), lambda e, h, st, cn: (e, 0, h)),
            pl.BlockSpec((1, DIM, HC), lambda e, h, st, cn: (e, 0, h)),
            pl.BlockSpec((1, 1, HC), lambda e, h, st, cn: (e, 0, h)),
            pl.BlockSpec((1, HC, DIM), lambda e, h, st, cn: (e, h, 0)),
            pl.BlockSpec((1, 1, DIM), lambda e, h, st, cn: (e, 0, 0)),
        ],
        out_specs=pl.BlockSpec((PAD, DIM), lambda e, h, st, cn: (0, 0)),
    )
    return pl.pallas_call(
        _ffn_body,
        grid_spec=grid_spec,
        out_shape=jax.ShapeDtypeStruct((PAD, DIM), jnp.float32),
    )(start_w, cnt_w, x_pad,
      Wg, bg.reshape(E, 1, HID), Wi, bi.reshape(E, 1, HID),
      Wo, bo.reshape(E, 1, DIM))


def kernel(x, mapped_tokens, Wg, bg, Wi, bi, Wo, bo):
    Bv, Tv, C = x.shape
    xf = x.reshape(Bv * Tv, C)
    mt2d = mapped_tokens.astype(jnp.int32).reshape(MROW, MCOL)

    pos2d, start_w, cnt_w = _routing(mt2d)
    pos = pos2d.reshape(T)

    x_pad = _make_sc_row_perm(T, PAD, DIM, scatter=True)(xf, pos)
    out_pad = _grouped_ffn(start_w, cnt_w, x_pad, Wg, bg, Wi, bi, Wo, bo)
    out = _make_sc_row_perm(T, T, DIM, scatter=False)(out_pad, pos)
    return out.reshape(Bv, Tv, C)


# speedup vs baseline: 1.0912x; 1.0043x over previous
import functools

import jax
import jax.numpy as jnp
from jax import lax
from jax.experimental import pallas as pl
from jax.experimental.pallas import tpu as pltpu
from jax.experimental.pallas import tpu_sc as plsc

DIM = 768
HID = DIM * 4
E = 16
T = 2048
TILE = 256
ALIGN = 8
PAD = T + E * ALIGN + TILE - ALIGN
PAD = ((PAD + TILE - 1) // TILE) * TILE
NH = 2
HC = HID // NH
HC2 = HC // 2
MROW = 16
MCOL = T // MROW


def _routing_body(mt_ref, pos_ref, start_ref, cnt_ref):
    mt = mt_ref[...]
    acc = jnp.zeros((MROW, MCOL), jnp.int32)
    ps = jnp.int32(0)
    for e in range(E):
        m = (mt == e).astype(jnp.int32)
        p = m
        s = 1
        while s < MCOL:
            p = p + jnp.concatenate(
                [jnp.zeros((MROW, s), jnp.int32), p[:, :MCOL - s]], axis=1)
            s *= 2
        rt = p[:, MCOL - 1:MCOL]
        q = rt
        s = 1
        while s < MROW:
            q = q + jnp.concatenate(
                [jnp.zeros((s, 1), jnp.int32), q[:MROW - s, :]], axis=0)
            s *= 2
        rank = p + (q - rt)
        acc = acc + m * (ps + rank - 1)
        cnt = jnp.sum(m)
        start_ref[e] = ps
        cnt_ref[e] = cnt
        ps = ps + ((cnt + ALIGN - 1) // ALIGN) * ALIGN
    pos_ref[...] = acc


def _routing(mt2d):
    return pl.pallas_call(
        _routing_body,
        out_shape=[
            jax.ShapeDtypeStruct((MROW, MCOL), jnp.int32),
            jax.ShapeDtypeStruct((E,), jnp.int32),
            jax.ShapeDtypeStruct((E,), jnp.int32),
        ],
        out_specs=[
            pl.BlockSpec(memory_space=pltpu.MemorySpace.VMEM),
            pl.BlockSpec(memory_space=pltpu.SMEM),
            pl.BlockSpec(memory_space=pltpu.SMEM),
        ],
        in_specs=[pl.BlockSpec(memory_space=pltpu.MemorySpace.VMEM)],
    )(mt2d)


def _make_sc_row_perm(n_src, n_dst, dim, scatter):
    info = plsc.get_sparse_core_info()
    nc, ns = info.num_cores, info.num_subcores
    nw = nc * ns
    per_w = n_src // nw
    mesh = plsc.VectorSubcoreMesh(core_axis_name="c", subcore_axis_name="s")

    @functools.partial(
        pl.kernel,
        mesh=mesh,
        out_type=jax.ShapeDtypeStruct((n_dst, dim), jnp.float32),
        scratch_types=[
            pltpu.VMEM((per_w,), jnp.int32),
            pltpu.VMEM((per_w, dim), jnp.float32),
            pltpu.SemaphoreType.DMA,
        ],
    )
    def perm_k(rows_hbm, idx_hbm, out_hbm, idx_v, rows_v, sem):
        wid = lax.axis_index("s") * nc + lax.axis_index("c")
        base = wid * per_w
        pltpu.sync_copy(idx_hbm.at[pl.ds(base, per_w)], idx_v)
        if scatter:
            pltpu.sync_copy(rows_hbm.at[pl.ds(base, per_w)], rows_v)
            pltpu.async_copy(rows_v, out_hbm.at[idx_v], sem).wait()
        else:
            pltpu.async_copy(rows_hbm.at[idx_v], rows_v, sem).wait()
            pltpu.sync_copy(rows_v, out_hbm.at[pl.ds(base, per_w)])

    return perm_k


def _ffn_body(start_ref, cnt_ref,
              x_ref, wg_ref, bg_ref, wi_ref, bi_ref, wo_ref, bo_ref,
              out_ref):
    e = pl.program_id(0)
    h = pl.program_id(1)
    start = start_ref[e]
    cnt = cnt_ref[e]
    ntiles = (cnt + TILE - 1) // TILE
    wg = wg_ref[0]
    wi = wi_ref[0]
    wo = wo_ref[0]
    bg = bg_ref[0, 0]
    bi = bi_ref[0, 0]
    bo = bo_ref[0, 0]

    def tile_body(k, _):
        base = pl.multiple_of(start, ALIGN) + k * TILE
        xb = x_ref[pl.ds(base, TILE), :].astype(jnp.bfloat16)
        wgb = wg.astype(jnp.bfloat16)
        wib = wi.astype(jnp.bfloat16)
        wob = wo.astype(jnp.bfloat16)
        g = jnp.dot(xb, wgb, preferred_element_type=jnp.float32) + bg
        i_ = jnp.dot(xb, wib, preferred_element_type=jnp.float32) + bi
        hdn = ((g * jax.nn.sigmoid(g)) * i_).astype(jnp.bfloat16)
        o = jnp.dot(hdn, wob, preferred_element_type=jnp.float32)
        rows = k * TILE + lax.broadcasted_iota(jnp.int32, (TILE, 1), 0)
        mask = rows < cnt

        @pl.when(h == 0)
        def _first():
            out_ref[pl.ds(base, TILE), :] = jnp.where(mask, o + bo, 0.0)

        @pl.when(h != 0)
        def _acc():
            out_ref[pl.ds(base, TILE), :] += jnp.where(mask, o, 0.0)

        return 0

    lax.fori_loop(0, ntiles, tile_body, 0)


def _grouped_ffn(start_w, cnt_w, x_pad, Wg, bg, Wi, bi, Wo, bo):
    grid_spec = pltpu.PrefetchScalarGridSpec(
        num_scalar_prefetch=2,
        grid=(E, NH),
        in_specs=[
            pl.BlockSpec((PAD, DIM), lambda e, h, st, cn: (0, 0)),
            pl.BlockSpec((1, DIM, HC), lambda e, h, st, cn: (e, 0, h)),
            pl.BlockSpec((1, 1, HC), lambda e, h, st, cn: (e, 0, h)),
            pl.BlockSpec((1, DIM, HC), lambda e, h, st, cn: (e, 0, h)),
            pl.BlockSpec((1, 1, HC), lambda e, h, st, cn: (e, 0, h)),
            pl.BlockSpec((1, HC, DIM), lambda e, h, st, cn: (e, h, 0)),
            pl.BlockSpec((1, 1, DIM), lambda e, h, st, cn: (e, 0, 0)),
        ],
        out_specs=pl.BlockSpec((PAD, DIM), lambda e, h, st, cn: (0, 0)),
    )
    return pl.pallas_call(
        _ffn_body,
        grid_spec=grid_spec,
        out_shape=jax.ShapeDtypeStruct((PAD, DIM), jnp.float32),
    )(start_w, cnt_w, x_pad,
      Wg, bg.reshape(E, 1, HID), Wi, bi.reshape(E, 1, HID),
      Wo, bo.reshape(E, 1, DIM))


def kernel(x, mapped_tokens, Wg, bg, Wi, bi, Wo, bo):
    Bv, Tv, C = x.shape
    xf = x.reshape(Bv * Tv, C)
    mt2d = mapped_tokens.astype(jnp.int32).reshape(MROW, MCOL)

    pos2d, start_w, cnt_w = _routing(mt2d)
    pos = pos2d.reshape(T)

    x_pad = _make_sc_row_perm(T, PAD, DIM, scatter=True)(xf, pos)
    out_pad = _grouped_ffn(start_w, cnt_w, x_pad, Wg, bg, Wi, bi, Wo, bo)
    out = _make_sc_row_perm(T, T, DIM, scatter=False)(out_pad, pos)
    return out.reshape(Bv, Tv, C)
